# baseline (speedup 1.0000x reference)
import functools

import jax
import jax.numpy as jnp
from jax import lax
from jax.experimental import pallas as pl
from jax.experimental.pallas import tpu as pltpu
from jax.experimental.pallas import tpu_sc as plsc

BATCH = 4096
HIST = 50
EMBED_DIM = 32
TOTAL = BATCH * HIST

_INFO = plsc.get_sparse_core_info()
NC = _INFO.num_cores
NS = _INFO.num_subcores
NW = NC * NS
B_PER_W = TOTAL // NW
CHUNK = 128
N_CHUNKS = B_PER_W // CHUNK
NBUF = 10
ROUNDS = N_CHUNKS // NBUF
DRAIN_SLACK = 2

_MESH = plsc.VectorSubcoreMesh(core_axis_name="c", subcore_axis_name="s")


@functools.partial(
    pl.kernel,
    out_type=jax.ShapeDtypeStruct((TOTAL, EMBED_DIM), jnp.float32),
    mesh=_MESH,
    scratch_types=[
        pltpu.VMEM((N_CHUNKS, CHUNK), jnp.int32),
        pltpu.VMEM((NBUF, CHUNK, EMBED_DIM), jnp.float32),
        pltpu.SemaphoreType.DMA((NBUF,)),
        pltpu.SemaphoreType.DMA((NBUF,)),
    ],
    compiler_params=pltpu.CompilerParams(use_tc_tiling_on_sc=False),
)
def _sc_gather(idx_hbm, table_hbm, out_hbm, idx_v, rows_v, sem_g, sem_o):
    wid = lax.axis_index("s") * NC + lax.axis_index("c")
    pltpu.sync_copy(idx_hbm.at[wid], idx_v)

    def gather(j, b):
        return pltpu.async_copy(
            table_hbm.at[idx_v.at[j]], rows_v.at[b], sem_g.at[b]
        )

    base = wid * B_PER_W

    def copy_out(j, b):
        return pltpu.make_async_copy(
            rows_v.at[b],
            out_hbm.at[pl.ds(base + j * CHUNK, CHUNK)],
            sem_o.at[b],
        )

    for b in range(NBUF):
        gather(b, b)

    @pl.loop(0, ROUNDS)
    def _(r):
        for b in range(NBUF):
            j = r * NBUF + b
            pltpu.make_async_copy(
                table_hbm.at[idx_v.at[j]], rows_v.at[b], sem_g.at[b]
            ).wait()
            copy_out(j, b).start()
            bn = (b - DRAIN_SLACK) % NBUF
            jo = r * NBUF + b - DRAIN_SLACK
            jn = jo + NBUF

            @pl.when((jo >= 0) & (jn < N_CHUNKS))
            def _():
                copy_out(jo, bn).wait()
                gather(jn, bn)

    for b in range(NBUF):
        j = N_CHUNKS - NBUF + b
        copy_out(j, b).wait()


def kernel(indices, embedding):
    idx = indices.astype(jnp.int32).reshape(NW, N_CHUNKS, CHUNK)
    out = _sc_gather(idx, embedding)
    return out.reshape(BATCH, HIST, EMBED_DIM)


# speedup vs baseline: 1.1365x; 1.1365x over previous
import functools

import jax
import jax.numpy as jnp
from jax import lax
from jax.experimental import pallas as pl
from jax.experimental.pallas import tpu as pltpu
from jax.experimental.pallas import tpu_sc as plsc

BATCH = 4096
HIST = 50
EMBED_DIM = 32

_INFO = plsc.get_sparse_core_info()
NC = _INFO.num_cores
NS = _INFO.num_subcores
NW = NC * NS
CHUNK = BATCH // NW
N_CHUNKS = HIST
NBUF = 5
ROUNDS = N_CHUNKS // NBUF
DRAIN_SLACK = 2
LANES = 16

_MESH = plsc.VectorSubcoreMesh(core_axis_name="c", subcore_axis_name="s")


@functools.partial(
    pl.kernel,
    out_type=jax.ShapeDtypeStruct((HIST, EMBED_DIM, BATCH), jnp.float32),
    mesh=_MESH,
    scratch_types=[
        pltpu.VMEM((N_CHUNKS, CHUNK), jnp.int32),
        pltpu.VMEM((NBUF, CHUNK, EMBED_DIM), jnp.float32),
        pltpu.VMEM((NBUF, EMBED_DIM, CHUNK), jnp.float32),
        pltpu.SemaphoreType.DMA((NBUF,)),
        pltpu.SemaphoreType.DMA((NBUF,)),
    ],
    compiler_params=pltpu.CompilerParams(
        use_tc_tiling_on_sc=False, needs_layout_passes=False
    ),
)
def _sc_gather(idx_hbm, table_hbm, out_hbm, idx_v, rows_v, trans_v, sem_g, sem_o):
    wid = lax.axis_index("s") * NC + lax.axis_index("c")
    b0 = wid * CHUNK
    pltpu.sync_copy(idx_hbm.at[:, pl.ds(b0, CHUNK)], idx_v)

    def gather(j, b):
        return pltpu.async_copy(
            table_hbm.at[idx_v.at[j]], rows_v.at[b], sem_g.at[b]
        )

    def copy_out(j, b):
        return pltpu.make_async_copy(
            trans_v.at[b],
            out_hbm.at[j, :, pl.ds(b0, CHUNK)],
            sem_o.at[b],
        )

    def transpose(b):
        rows = rows_v.at[b]
        trans = trans_v.at[b]

        @pl.loop(0, EMBED_DIM)
        def _(e):
            col = jnp.full((LANES,), e, jnp.int32)
            for g in range(CHUNK // LANES):
                base = lax.iota(jnp.int32, LANES) + g * LANES
                v = plsc.load_gather(rows, [base, col])
                trans[e, pl.ds(g * LANES, LANES)] = v

    for b in range(NBUF):
        gather(b, b)

    @pl.loop(0, ROUNDS)
    def _(r):
        for b in range(NBUF):
            j = r * NBUF + b
            pltpu.make_async_copy(
                table_hbm.at[idx_v.at[j]], rows_v.at[b], sem_g.at[b]
            ).wait()
            transpose(b)
            copy_out(j, b).start()
            bn = (b - DRAIN_SLACK) % NBUF
            jo = r * NBUF + b - DRAIN_SLACK
            jn = jo + NBUF

            @pl.when((jo >= 0) & (jn < N_CHUNKS))
            def _():
                copy_out(jo, bn).wait()
                gather(jn, bn)

    for b in range(NBUF):
        j = N_CHUNKS - NBUF + b
        copy_out(j, b).wait()


def kernel(indices, embedding):
    idx_t = jnp.transpose(indices.astype(jnp.int32))
    out = _sc_gather(idx_t, embedding)
    return jnp.transpose(out, (2, 0, 1))
